# baseline (speedup 1.0000x reference)
import jax
import jax.numpy as jnp
from jax import lax
from jax.experimental import pallas as pl
from jax.experimental.pallas import tpu as pltpu
from jax.experimental.pallas import tpu_sc as plsc

NC, NS, L = 2, 16, 16
NW = NC * NS

P = 100000
PPAD = 102400
PTS_PER_W = PPAD // NW
E = 6400000
E_PER_W = E // NW
CHUNK = 1600
NCHUNKS = E_PER_W // CHUNK
GROUPS = CHUNK // L
IDX_GRP = 128
NFULL = CHUNK // IDX_GRP
REM = CHUNK - NFULL * IDX_GRP
EPS2 = 1e-24


def _wid():
    return lax.axis_index("c") * NS + lax.axis_index("s")


def _col(c):
    return jnp.full((L,), c, dtype=jnp.int32)


def _build_table(pred4, nidx, gt8, t_hbm, predv, nidxv, gv, tv, sem):
    base = _wid() * PTS_PER_W
    pltpu.sync_copy(pred4.at[pl.ds(base, PTS_PER_W), :], predv)
    pltpu.sync_copy(nidx.at[pl.ds(base, PTS_PER_W)], nidxv)
    cps = []
    for g in range(PTS_PER_W // IDX_GRP):
        s = pl.ds(g * IDX_GRP, IDX_GRP)
        cps.append(pltpu.async_copy(gt8.at[nidxv.at[s]], gv.at[s, :], sem))
    for cp in cps:
        cp.wait()

    def body(i, _):
        rows = i * L + lax.iota(jnp.int32, L)
        gid = rows + base
        px = plsc.load_gather(predv, [rows, _col(0)])
        py = plsc.load_gather(predv, [rows, _col(1)])
        pz = plsc.load_gather(predv, [rows, _col(2)])
        gx = plsc.load_gather(gv, [rows, _col(0)])
        gy = plsc.load_gather(gv, [rows, _col(1)])
        gz = plsc.load_gather(gv, [rows, _col(2)])
        inv = 1.0 / jnp.maximum(gx * gx + gy * gy + gz * gz, EPS2)
        mark = jnp.where(gid != 0, 1.0, 0.0).astype(jnp.float32)
        plsc.store_scatter(tv, [rows, _col(0)], px)
        plsc.store_scatter(tv, [rows, _col(1)], py)
        plsc.store_scatter(tv, [rows, _col(2)], pz)
        plsc.store_scatter(tv, [rows, _col(3)], gx)
        plsc.store_scatter(tv, [rows, _col(4)], gy)
        plsc.store_scatter(tv, [rows, _col(5)], gz)
        plsc.store_scatter(tv, [rows, _col(6)], inv)
        plsc.store_scatter(tv, [rows, _col(7)], mark)
        return 0

    lax.fori_loop(0, PTS_PER_W // L, body, 0)
    pltpu.sync_copy(tv, t_hbm.at[pl.ds(base, PTS_PER_W), :])


def _edge_loss(t_hbm, el, lossp, cntp,
               ia, ib, ria, rja, rib, rjb, lv, cv,
               isem0, isem1, gsem0, gsem1):
    wid = _wid()
    ebase = wid * E_PER_W

    ibufs = ((ia, isem0), (ib, isem1))
    gbufs = ((ria, rja, gsem0), (rib, rjb, gsem1))

    def slices():
        for g in range(NFULL + 1):
            n = IDX_GRP if g < NFULL else REM
            yield pl.ds(g * IDX_GRP, n)

    def _off(k):
        return ebase + jnp.minimum(k, NCHUNKS - 1) * CHUNK

    def fire_idx(k, ibuf):
        iv, isem = ibuf
        pltpu.async_copy(el.at[:, pl.ds(_off(k), CHUNK)], iv, isem)

    def drain_idx(ibuf):
        iv, isem = ibuf
        pltpu.make_async_copy(el.at[:, pl.ds(ebase, CHUNK)], iv, isem).wait()

    def fire_gather(ibuf, gbuf):
        iv, _ = ibuf
        ri, rj, gsem = gbuf
        for s in slices():
            pltpu.async_copy(t_hbm.at[iv.at[0, s]], ri.at[s, :], gsem)
            pltpu.async_copy(t_hbm.at[iv.at[1, s]], rj.at[s, :], gsem)

    def drain_gather(ibuf, gbuf):
        iv, _ = ibuf
        ri, rj, gsem = gbuf
        for s in slices():
            pltpu.make_async_copy(t_hbm.at[iv.at[0, s]], ri.at[s, :], gsem).wait()
            pltpu.make_async_copy(t_hbm.at[iv.at[1, s]], rj.at[s, :], gsem).wait()

    def compute(gbuf, carry):
        ri, rj, _ = gbuf

        def grp(i, c2):
            al, ac = c2
            rows = i * L + lax.iota(jnp.int32, L)
            pxi = plsc.load_gather(ri, [rows, _col(0)])
            pyi = plsc.load_gather(ri, [rows, _col(1)])
            pzi = plsc.load_gather(ri, [rows, _col(2)])
            gxi = plsc.load_gather(ri, [rows, _col(3)])
            gyi = plsc.load_gather(ri, [rows, _col(4)])
            gzi = plsc.load_gather(ri, [rows, _col(5)])
            inv = plsc.load_gather(ri, [rows, _col(6)])
            mi = plsc.load_gather(ri, [rows, _col(7)])
            pxj = plsc.load_gather(rj, [rows, _col(0)])
            pyj = plsc.load_gather(rj, [rows, _col(1)])
            pzj = plsc.load_gather(rj, [rows, _col(2)])
            mj = plsc.load_gather(rj, [rows, _col(7)])
            dx = pxi - pxj
            dy = pyi - pyj
            dz = pzi - pzj
            dd = dx * dx + dy * dy + dz * dz
            dt = dx * gxi + dy * gyi + dz * gzi
            loss = dt * dt * inv / jnp.maximum(dd, EPS2)
            m = jnp.maximum(mi, mj)
            return (al + loss * m, ac + m)

        return lax.fori_loop(0, GROUPS, grp, carry)

    zero = jnp.zeros((L,), jnp.float32)
    fire_idx(0, ibufs[0])
    fire_idx(1, ibufs[1])
    drain_idx(ibufs[0])
    fire_gather(ibufs[0], gbufs[0])

    def pair(t, carry):
        c0 = 2 * t
        drain_idx(ibufs[1])
        fire_gather(ibufs[1], gbufs[1])
        fire_idx(c0 + 2, ibufs[0])
        drain_gather(ibufs[0], gbufs[0])
        carry = compute(gbufs[0], carry)
        drain_idx(ibufs[0])
        fire_gather(ibufs[0], gbufs[0])
        fire_idx(c0 + 3, ibufs[1])
        drain_gather(ibufs[1], gbufs[1])
        carry = compute(gbufs[1], carry)
        return carry

    accl, accc = lax.fori_loop(0, (NCHUNKS - 1) // 2, pair, (zero, zero))
    drain_idx(ibufs[1])
    drain_gather(ibufs[0], gbufs[0])
    accl, accc = compute(gbufs[0], (accl, accc))
    lv[...] = accl
    cv[...] = accc
    pltpu.sync_copy(lv, lossp.at[wid])
    pltpu.sync_copy(cv, cntp.at[wid])


def kernel(pred, nearest_gt_idx, gt_normals, edge_list):
    pred4 = jnp.zeros((PPAD, 4), jnp.float32).at[:P, :3].set(pred)
    nidx = jnp.zeros((PPAD,), jnp.int32).at[:P].set(nearest_gt_idx[0])
    gt8 = jnp.zeros((gt_normals.shape[1], 8), jnp.float32).at[:, :3].set(
        gt_normals[0])
    mesh = plsc.VectorSubcoreMesh(
        core_axis_name="c", subcore_axis_name="s",
        num_cores=NC, num_subcores=NS)

    params = pltpu.CompilerParams(
        needs_layout_passes=False, use_tc_tiling_on_sc=False)

    t = pl.kernel(
        _build_table,
        out_type=jax.ShapeDtypeStruct((PPAD, 8), jnp.float32),
        mesh=mesh,
        compiler_params=params,
        scratch_types=[
            pltpu.VMEM((PTS_PER_W, 4), jnp.float32),
            pltpu.VMEM((PTS_PER_W,), jnp.int32),
            pltpu.VMEM((PTS_PER_W, 8), jnp.float32),
            pltpu.VMEM((PTS_PER_W, 8), jnp.float32),
            pltpu.SemaphoreType.DMA,
        ],
    )(pred4, nidx, gt8)

    lossp, cntp = pl.kernel(
        _edge_loss,
        out_type=[
            jax.ShapeDtypeStruct((NW, L), jnp.float32),
            jax.ShapeDtypeStruct((NW, L), jnp.float32),
        ],
        mesh=mesh,
        compiler_params=params,
        scratch_types=[
            pltpu.VMEM((2, CHUNK), jnp.int32),
            pltpu.VMEM((2, CHUNK), jnp.int32),
            pltpu.VMEM((CHUNK, 8), jnp.float32),
            pltpu.VMEM((CHUNK, 8), jnp.float32),
            pltpu.VMEM((CHUNK, 8), jnp.float32),
            pltpu.VMEM((CHUNK, 8), jnp.float32),
            pltpu.VMEM((L,), jnp.float32),
            pltpu.VMEM((L,), jnp.float32),
            pltpu.SemaphoreType.DMA,
            pltpu.SemaphoreType.DMA,
            pltpu.SemaphoreType.DMA,
            pltpu.SemaphoreType.DMA,
        ],
    )(t, edge_list)

    return jnp.sum(lossp) / jnp.sum(cntp)


# speedup vs baseline: 1.0000x; 1.0000x over previous
import jax
import jax.numpy as jnp
from jax import lax
from jax.experimental import pallas as pl
from jax.experimental.pallas import tpu as pltpu
from jax.experimental.pallas import tpu_sc as plsc

NC, NS, L = 2, 16, 16
NW = NC * NS

P = 100000
PPAD = 102400
PTS_PER_W = PPAD // NW
E = 6400000
E_PER_W = E // NW
CHUNK = 1600
NCHUNKS = E_PER_W // CHUNK
GROUPS = CHUNK // L
IDX_GRP = 128
NFULL = CHUNK // IDX_GRP
REM = CHUNK - NFULL * IDX_GRP
EPS2 = 1e-24


def _wid():
    return lax.axis_index("c") * NS + lax.axis_index("s")


def _col(c):
    return jnp.full((L,), c, dtype=jnp.int32)


def _build_table(pred4, nidx, gt8, t_hbm, predv, nidxv, gv, tv, sem):
    base = _wid() * PTS_PER_W
    pltpu.sync_copy(pred4.at[pl.ds(base, PTS_PER_W), :], predv)
    pltpu.sync_copy(nidx.at[pl.ds(base, PTS_PER_W)], nidxv)
    cps = []
    for g in range(PTS_PER_W // IDX_GRP):
        s = pl.ds(g * IDX_GRP, IDX_GRP)
        cps.append(pltpu.async_copy(gt8.at[nidxv.at[s]], gv.at[s, :], sem))
    for cp in cps:
        cp.wait()

    def body(i, _):
        rows = i * L + lax.iota(jnp.int32, L)
        gid = rows + base
        px = plsc.load_gather(predv, [rows, _col(0)])
        py = plsc.load_gather(predv, [rows, _col(1)])
        pz = plsc.load_gather(predv, [rows, _col(2)])
        gx = plsc.load_gather(gv, [rows, _col(0)])
        gy = plsc.load_gather(gv, [rows, _col(1)])
        gz = plsc.load_gather(gv, [rows, _col(2)])
        inv = 1.0 / jnp.maximum(gx * gx + gy * gy + gz * gz, EPS2)
        mark = jnp.where(gid != 0, 1.0, 0.0).astype(jnp.float32)
        plsc.store_scatter(tv, [rows, _col(0)], px)
        plsc.store_scatter(tv, [rows, _col(1)], py)
        plsc.store_scatter(tv, [rows, _col(2)], pz)
        plsc.store_scatter(tv, [rows, _col(3)], gx)
        plsc.store_scatter(tv, [rows, _col(4)], gy)
        plsc.store_scatter(tv, [rows, _col(5)], gz)
        plsc.store_scatter(tv, [rows, _col(6)], inv)
        plsc.store_scatter(tv, [rows, _col(7)], mark)
        return 0

    lax.fori_loop(0, PTS_PER_W // L, body, 0)
    pltpu.sync_copy(tv, t_hbm.at[pl.ds(base, PTS_PER_W), :])


def _edge_loss(t_hbm, el, lossp, cntp,
               i0, i1, i2, ri0, rj0, ri1, rj1, ri2, rj2, lv, cv,
               is0, is1, is2, gs0, gs1, gs2):
    wid = _wid()
    ebase = wid * E_PER_W

    S = ((i0, ri0, rj0, is0, gs0),
         (i1, ri1, rj1, is1, gs1),
         (i2, ri2, rj2, is2, gs2))

    def slices():
        for g in range(NFULL + 1):
            n = IDX_GRP if g < NFULL else REM
            yield pl.ds(g * IDX_GRP, n)

    def _off(k):
        return ebase + jnp.minimum(k, NCHUNKS - 1) * CHUNK

    def fire_idx(k, s_):
        iv, _, _, isem, _ = s_
        pltpu.async_copy(el.at[:, pl.ds(_off(k), CHUNK)], iv, isem)

    def drain_idx(s_):
        iv, _, _, isem, _ = s_
        pltpu.make_async_copy(el.at[:, pl.ds(ebase, CHUNK)], iv, isem).wait()

    def fire_gather(s_):
        iv, ri, rj, _, gsem = s_
        for s in slices():
            pltpu.async_copy(t_hbm.at[iv.at[0, s]], ri.at[s, :], gsem)
            pltpu.async_copy(t_hbm.at[iv.at[1, s]], rj.at[s, :], gsem)

    def drain_gather(s_):
        iv, ri, rj, _, gsem = s_
        for s in slices():
            pltpu.make_async_copy(t_hbm.at[iv.at[0, s]], ri.at[s, :], gsem).wait()
            pltpu.make_async_copy(t_hbm.at[iv.at[1, s]], rj.at[s, :], gsem).wait()

    def compute(s_, carry):
        _, ri, rj, _, _ = s_

        def grp(i, c2):
            al, ac = c2
            rows = i * L + lax.iota(jnp.int32, L)
            pxi = plsc.load_gather(ri, [rows, _col(0)])
            pyi = plsc.load_gather(ri, [rows, _col(1)])
            pzi = plsc.load_gather(ri, [rows, _col(2)])
            gxi = plsc.load_gather(ri, [rows, _col(3)])
            gyi = plsc.load_gather(ri, [rows, _col(4)])
            gzi = plsc.load_gather(ri, [rows, _col(5)])
            inv = plsc.load_gather(ri, [rows, _col(6)])
            mi = plsc.load_gather(ri, [rows, _col(7)])
            pxj = plsc.load_gather(rj, [rows, _col(0)])
            pyj = plsc.load_gather(rj, [rows, _col(1)])
            pzj = plsc.load_gather(rj, [rows, _col(2)])
            mj = plsc.load_gather(rj, [rows, _col(7)])
            dx = pxi - pxj
            dy = pyi - pyj
            dz = pzi - pzj
            dd = dx * dx + dy * dy + dz * dz
            dt = dx * gxi + dy * gyi + dz * gzi
            loss = dt * dt * inv / jnp.maximum(dd, EPS2)
            m = jnp.maximum(mi, mj)
            return (al + loss * m, ac + m)

        return lax.fori_loop(0, GROUPS, grp, carry)

    zero = jnp.zeros((L,), jnp.float32)
    fire_idx(0, S[0])
    fire_idx(1, S[1])
    fire_idx(2, S[2])
    drain_idx(S[0])
    fire_gather(S[0])
    drain_idx(S[1])
    fire_gather(S[1])

    def tri(t, carry):
        c0 = 3 * t
        for j, (a, b) in enumerate(((0, 2), (1, 0), (2, 1))):
            c = c0 + j
            sc, sc2 = S[a], S[b]
            drain_idx(sc2)
            fire_gather(sc2)
            drain_gather(sc)
            fire_idx(c + 3, sc)
            carry = compute(sc, carry)
        return carry

    accl, accc = lax.fori_loop(0, (NCHUNKS - 2) // 3, tri, (zero, zero))
    drain_idx(S[2])
    drain_gather(S[0])
    accl, accc = compute(S[0], (accl, accc))
    drain_gather(S[1])
    accl, accc = compute(S[1], (accl, accc))
    lv[...] = accl
    cv[...] = accc
    pltpu.sync_copy(lv, lossp.at[wid])
    pltpu.sync_copy(cv, cntp.at[wid])


def kernel(pred, nearest_gt_idx, gt_normals, edge_list):
    pred4 = jnp.zeros((PPAD, 4), jnp.float32).at[:P, :3].set(pred)
    nidx = jnp.zeros((PPAD,), jnp.int32).at[:P].set(nearest_gt_idx[0])
    gt8 = jnp.zeros((gt_normals.shape[1], 8), jnp.float32).at[:, :3].set(
        gt_normals[0])
    mesh = plsc.VectorSubcoreMesh(
        core_axis_name="c", subcore_axis_name="s",
        num_cores=NC, num_subcores=NS)

    params = pltpu.CompilerParams(
        needs_layout_passes=False, use_tc_tiling_on_sc=False)

    t = pl.kernel(
        _build_table,
        out_type=jax.ShapeDtypeStruct((PPAD, 8), jnp.float32),
        mesh=mesh,
        compiler_params=params,
        scratch_types=[
            pltpu.VMEM((PTS_PER_W, 4), jnp.float32),
            pltpu.VMEM((PTS_PER_W,), jnp.int32),
            pltpu.VMEM((PTS_PER_W, 8), jnp.float32),
            pltpu.VMEM((PTS_PER_W, 8), jnp.float32),
            pltpu.SemaphoreType.DMA,
        ],
    )(pred4, nidx, gt8)

    lossp, cntp = pl.kernel(
        _edge_loss,
        out_type=[
            jax.ShapeDtypeStruct((NW, L), jnp.float32),
            jax.ShapeDtypeStruct((NW, L), jnp.float32),
        ],
        mesh=mesh,
        compiler_params=params,
        scratch_types=[
            pltpu.VMEM((2, CHUNK), jnp.int32),
            pltpu.VMEM((2, CHUNK), jnp.int32),
            pltpu.VMEM((2, CHUNK), jnp.int32),
            pltpu.VMEM((CHUNK, 8), jnp.float32),
            pltpu.VMEM((CHUNK, 8), jnp.float32),
            pltpu.VMEM((CHUNK, 8), jnp.float32),
            pltpu.VMEM((CHUNK, 8), jnp.float32),
            pltpu.VMEM((CHUNK, 8), jnp.float32),
            pltpu.VMEM((CHUNK, 8), jnp.float32),
            pltpu.VMEM((L,), jnp.float32),
            pltpu.VMEM((L,), jnp.float32),
            pltpu.SemaphoreType.DMA,
            pltpu.SemaphoreType.DMA,
            pltpu.SemaphoreType.DMA,
            pltpu.SemaphoreType.DMA,
            pltpu.SemaphoreType.DMA,
            pltpu.SemaphoreType.DMA,
        ],
    )(t, edge_list)

    return jnp.sum(lossp) / jnp.sum(cntp)


# speedup vs baseline: 1.0102x; 1.0101x over previous
import jax
import jax.numpy as jnp
from jax import lax
from jax.experimental import pallas as pl
from jax.experimental.pallas import tpu as pltpu
from jax.experimental.pallas import tpu_sc as plsc

NC, NS, L = 2, 16, 16
NW = NC * NS

P = 100000
PPAD = 102400
PTS_PER_W = PPAD // NW
E = 6400000
E_PER_W = E // NW
CHUNK = 1600
NCHUNKS = E_PER_W // CHUNK
GROUPS = CHUNK // L
IDX_GRP = 128
NFULL = CHUNK // IDX_GRP
REM = CHUNK - NFULL * IDX_GRP
EPS2 = 1e-24


def _wid():
    return lax.axis_index("c") * NS + lax.axis_index("s")


def _col(c):
    return jnp.full((L,), c, dtype=jnp.int32)


def _build_table(pred4, nidx, gt8, t_hbm, predv, nidxv, gv, tv, sem):
    base = _wid() * PTS_PER_W
    pltpu.sync_copy(pred4.at[pl.ds(base, PTS_PER_W), :], predv)
    pltpu.sync_copy(nidx.at[pl.ds(base, PTS_PER_W)], nidxv)
    cps = []
    for g in range(PTS_PER_W // IDX_GRP):
        s = pl.ds(g * IDX_GRP, IDX_GRP)
        cps.append(pltpu.async_copy(gt8.at[nidxv.at[s]], gv.at[s, :], sem))
    for cp in cps:
        cp.wait()

    def body(i, _):
        rows = i * L + lax.iota(jnp.int32, L)
        px = plsc.load_gather(predv, [rows, _col(0)])
        py = plsc.load_gather(predv, [rows, _col(1)])
        pz = plsc.load_gather(predv, [rows, _col(2)])
        gx = plsc.load_gather(gv, [rows, _col(0)])
        gy = plsc.load_gather(gv, [rows, _col(1)])
        gz = plsc.load_gather(gv, [rows, _col(2)])
        inv = 1.0 / jnp.maximum(gx * gx + gy * gy + gz * gz, EPS2)
        plsc.store_scatter(tv, [rows, _col(0)], px)
        plsc.store_scatter(tv, [rows, _col(1)], py)
        plsc.store_scatter(tv, [rows, _col(2)], pz)
        plsc.store_scatter(tv, [rows, _col(3)], gx)
        plsc.store_scatter(tv, [rows, _col(4)], gy)
        plsc.store_scatter(tv, [rows, _col(5)], gz)
        plsc.store_scatter(tv, [rows, _col(6)], inv)
        plsc.store_scatter(tv, [rows, _col(7)], jnp.zeros((L,), jnp.float32))
        return 0

    lax.fori_loop(0, PTS_PER_W // L, body, 0)
    pltpu.sync_copy(tv, t_hbm.at[pl.ds(base, PTS_PER_W), :])


def _edge_loss(t_hbm, e01, lossp, cntp,
               i0a, i1a, ria, rja, i0b, i1b, rib, rjb, lv, cv, sem0, sem1):
    wid = _wid()
    ebase = wid * E_PER_W
    bufs = ((i0a, i1a, ria, rja, sem0), (i0b, i1b, rib, rjb, sem1))

    def slices():
        for g in range(NFULL + 1):
            n = IDX_GRP if g < NFULL else REM
            yield pl.ds(g * IDX_GRP, n)

    def fire(c, buf):
        i0v, i1v, ri, rj, sem = buf
        off = ebase + c * CHUNK
        pltpu.sync_copy(e01.at[pl.ds(off, CHUNK)], i0v)
        pltpu.sync_copy(e01.at[pl.ds(E + off, CHUNK)], i1v)
        for s in slices():
            pltpu.async_copy(t_hbm.at[i0v.at[s]], ri.at[s, :], sem)
            pltpu.async_copy(t_hbm.at[i1v.at[s]], rj.at[s, :], sem)

    def drain(buf):
        i0v, i1v, ri, rj, sem = buf
        for s in slices():
            pltpu.make_async_copy(t_hbm.at[i0v.at[s]], ri.at[s, :], sem).wait()
            pltpu.make_async_copy(t_hbm.at[i1v.at[s]], rj.at[s, :], sem).wait()

    def compute(buf, carry):
        i0v, i1v, ri, rj, _ = buf

        def grp(i, c2):
            al, ac = c2
            rows = i * L + lax.iota(jnp.int32, L)
            pxi = plsc.load_gather(ri, [rows, _col(0)])
            pyi = plsc.load_gather(ri, [rows, _col(1)])
            pzi = plsc.load_gather(ri, [rows, _col(2)])
            gxi = plsc.load_gather(ri, [rows, _col(3)])
            gyi = plsc.load_gather(ri, [rows, _col(4)])
            gzi = plsc.load_gather(ri, [rows, _col(5)])
            inv = plsc.load_gather(ri, [rows, _col(6)])
            pxj = plsc.load_gather(rj, [rows, _col(0)])
            pyj = plsc.load_gather(rj, [rows, _col(1)])
            pzj = plsc.load_gather(rj, [rows, _col(2)])
            dx = pxi - pxj
            dy = pyi - pyj
            dz = pzi - pzj
            dd = dx * dx + dy * dy + dz * dz
            dt = dx * gxi + dy * gyi + dz * gzi
            loss = dt * dt * inv / jnp.maximum(dd, EPS2)
            i0 = i0v[pl.ds(i * L, L)]
            i1 = i1v[pl.ds(i * L, L)]
            m = (i0 != 0) | (i1 != 0)
            return (al + jnp.where(m, loss, 0.0),
                    ac + jnp.where(m, 1.0, 0.0))

        return lax.fori_loop(0, GROUPS, grp, carry)

    zero = jnp.zeros((L,), jnp.float32)
    fire(0, bufs[0])

    def pair(t, carry):
        c0 = 2 * t
        fire(c0 + 1, bufs[1])
        drain(bufs[0])
        carry = compute(bufs[0], carry)
        fire(c0 + 2, bufs[0])
        drain(bufs[1])
        carry = compute(bufs[1], carry)
        return carry

    accl, accc = lax.fori_loop(0, (NCHUNKS - 1) // 2, pair, (zero, zero))
    drain(bufs[0])
    accl, accc = compute(bufs[0], (accl, accc))
    lv[...] = accl
    cv[...] = accc
    pltpu.sync_copy(lv, lossp.at[wid])
    pltpu.sync_copy(cv, cntp.at[wid])


def kernel(pred, nearest_gt_idx, gt_normals, edge_list):
    pred4 = jnp.zeros((PPAD, 4), jnp.float32).at[:P, :3].set(pred)
    nidx = jnp.zeros((PPAD,), jnp.int32).at[:P].set(nearest_gt_idx[0])
    gt8 = jnp.zeros((gt_normals.shape[1], 8), jnp.float32).at[:, :3].set(
        gt_normals[0])
    e01 = jnp.reshape(edge_list, (-1,))
    mesh = plsc.VectorSubcoreMesh(
        core_axis_name="c", subcore_axis_name="s",
        num_cores=NC, num_subcores=NS)

    params = pltpu.CompilerParams(
        needs_layout_passes=False, use_tc_tiling_on_sc=False)

    t = pl.kernel(
        _build_table,
        out_type=jax.ShapeDtypeStruct((PPAD, 8), jnp.float32),
        mesh=mesh,
        compiler_params=params,
        scratch_types=[
            pltpu.VMEM((PTS_PER_W, 4), jnp.float32),
            pltpu.VMEM((PTS_PER_W,), jnp.int32),
            pltpu.VMEM((PTS_PER_W, 8), jnp.float32),
            pltpu.VMEM((PTS_PER_W, 8), jnp.float32),
            pltpu.SemaphoreType.DMA,
        ],
    )(pred4, nidx, gt8)

    lossp, cntp = pl.kernel(
        _edge_loss,
        out_type=[
            jax.ShapeDtypeStruct((NW, L), jnp.float32),
            jax.ShapeDtypeStruct((NW, L), jnp.float32),
        ],
        mesh=mesh,
        compiler_params=params,
        scratch_types=[
            pltpu.VMEM((CHUNK,), jnp.int32),
            pltpu.VMEM((CHUNK,), jnp.int32),
            pltpu.VMEM((CHUNK, 8), jnp.float32),
            pltpu.VMEM((CHUNK, 8), jnp.float32),
            pltpu.VMEM((CHUNK,), jnp.int32),
            pltpu.VMEM((CHUNK,), jnp.int32),
            pltpu.VMEM((CHUNK, 8), jnp.float32),
            pltpu.VMEM((CHUNK, 8), jnp.float32),
            pltpu.VMEM((L,), jnp.float32),
            pltpu.VMEM((L,), jnp.float32),
            pltpu.SemaphoreType.DMA,
            pltpu.SemaphoreType.DMA,
        ],
    )(t, e01)

    return jnp.sum(lossp) / jnp.sum(cntp)
